# baseline (speedup 1.0000x reference)
import functools

import jax
import jax.numpy as jnp
from jax import lax
from jax.experimental import pallas as pl
from jax.experimental.pallas import tpu as pltpu
from jax.experimental.pallas import tpu_sc as plsc

_N = 10000
_E = 320000
_LN = 320000
_LE = 640000
_H = 64
_NB = 64
_ANL = 5
_NJ = 3

_NC = 2
_NS = 16
_NW = _NC * _NS
_CH = 400

_mesh = plsc.VectorSubcoreMesh(core_axis_name="c", subcore_axis_name="s")
_sc_params = pltpu.CompilerParams(use_tc_tiling_on_sc=False)


def _p1_body(nch, A_hbm, B_hbm, d_hbm, s_hbm, ea_hbm, w_hbm,
             t_hbm, st_hbm, dv, sv, eav, wv, Ar, Br, stv, sem):
    wid = lax.axis_index("s") * _NC + lax.axis_index("c")
    base0 = wid * (nch * _CH)
    pltpu.sync_copy(w_hbm, wv)
    for j in range(8):
        stv[pl.ds(j * 16, 16)] = jnp.zeros((16,), jnp.float32)

    def chunk(i, carry):
        base = base0 + i * _CH
        pltpu.sync_copy(d_hbm.at[pl.ds(base, _CH)], dv)
        pltpu.sync_copy(s_hbm.at[pl.ds(base, _CH)], sv)
        pltpu.sync_copy(ea_hbm.at[pl.ds(base, _CH)], eav)
        pltpu.async_copy(A_hbm.at[dv], Ar, sem).wait()
        pltpu.async_copy(B_hbm.at[sv], Br, sem).wait()

        w0 = wv[pl.ds(0, 16)]
        w1 = wv[pl.ds(16, 16)]
        w2 = wv[pl.ds(32, 16)]
        w3 = wv[pl.ds(48, 16)]
        ws = (w0, w1, w2, w3)

        def grp(gi, acc):
            g16 = eav[pl.ds(gi * 16, 16)]
            out = list(acc)
            for j in range(16):
                e = gi * 16 + j
                eab = jnp.full((16,), g16[j])
                for c in range(4):
                    a = Ar[e, pl.ds(c * 16, 16)]
                    b = Br[e, pl.ds(c * 16, 16)]
                    t = a + b + eab * ws[c]
                    Ar[e, pl.ds(c * 16, 16)] = t
                    out[c] = out[c] + t
                    out[4 + c] = out[4 + c] + t * t
            return tuple(out)

        z = jnp.zeros((16,), jnp.float32)
        acc = lax.fori_loop(0, _CH // 16, grp, (z,) * 8)
        for c in range(8):
            stv[pl.ds(c * 16, 16)] = stv[pl.ds(c * 16, 16)] + acc[c]
        pltpu.sync_copy(Ar, t_hbm.at[pl.ds(base, _CH)])
        return carry

    lax.fori_loop(0, nch, chunk, 0)
    pltpu.sync_copy(stv, st_hbm.at[wid])


def _make_p1(ec):
    nch = ec // (_NW * _CH)
    return functools.partial(
        pl.kernel,
        functools.partial(_p1_body, nch),
        out_type=[jax.ShapeDtypeStruct((ec, _H), jnp.float32),
                  jax.ShapeDtypeStruct((_NW, 2 * _H), jnp.float32)],
        mesh=_mesh,
        scratch_types=[pltpu.VMEM((_CH,), jnp.int32),
                       pltpu.VMEM((_CH,), jnp.int32),
                       pltpu.VMEM((_CH,), jnp.float32),
                       pltpu.VMEM((_H,), jnp.float32),
                       pltpu.VMEM((_CH, _H), jnp.float32),
                       pltpu.VMEM((_CH, _H), jnp.float32),
                       pltpu.VMEM((2 * _H,), jnp.float32),
                       pltpu.SemaphoreType.DMA],
        compiler_params=_sc_params,
    )()


_p1_atom = _make_p1(_E)


def _p2_body(nch, nrows_pad, transform, rows_hbm, i_hbm, ab_hbm,
             out_hbm, iv2, rv, abv, sacc, sem):
    cid = lax.axis_index("c")
    sid = lax.axis_index("s")
    wid = sid * _NC + cid
    base0 = wid * (nch * _CH)
    garbage = nrows_pad - 1
    zrows = nrows_pad // _NS

    def zrow(r, carry):
        for c in range(4):
            rv[r, pl.ds(c * 16, 16)] = jnp.zeros((16,), jnp.float32)
        return carry
    lax.fori_loop(0, 512, zrow, 0)

    nfull = zrows // 512
    for r in range(nfull):
        pltpu.sync_copy(rv, sacc.at[pl.ds(sid * zrows + r * 512, 512)])
    rem = zrows - nfull * 512
    if rem:
        pltpu.sync_copy(rv.at[pl.ds(0, rem)],
                        sacc.at[pl.ds(sid * zrows + nfull * 512, rem)])
    plsc.subcore_barrier()

    if transform:
        pltpu.sync_copy(ab_hbm, abv)

    def chunk(i, carry):
        base = base0 + i * _CH
        for g in range(3):
            pltpu.sync_copy(i_hbm.at[pl.ds(base + g * 128, 128)], iv2.at[g])
        pltpu.sync_copy(i_hbm.at[pl.ds(base + 384, 16)],
                        iv2.at[3, pl.ds(0, 16)])
        for k in range(1, 8):
            iv2[3, pl.ds(k * 16, 16)] = jnp.full((16,), garbage, jnp.int32)
        pltpu.sync_copy(rows_hbm.at[pl.ds(base, _CH)], rv.at[pl.ds(0, _CH)])
        if transform:
            def row(e, rc):
                for c in range(4):
                    t = rv[e, pl.ds(c * 16, 16)]
                    z = t * abv[pl.ds(c * 16, 16)] + abv[pl.ds(64 + c * 16, 16)]
                    p = z / (1.0 + jnp.exp(-z))
                    rv[e, pl.ds(c * 16, 16)] = p
                return rc
            lax.fori_loop(0, _CH, row, 0)
        for g in range(4):
            pltpu.sync_copy(rv.at[pl.ds(g * 128, 128)],
                            sacc.at[iv2.at[g]], add=True)
        return carry

    lax.fori_loop(0, nch, chunk, 0)
    plsc.subcore_barrier()
    pltpu.sync_copy(sacc.at[pl.ds(sid * zrows, zrows)],
                    out_hbm.at[cid, pl.ds(sid * zrows, zrows)])


def _make_p2_small(ec, nrows, transform):
    nch = ec // (_NW * _CH)
    nrows_pad = ((nrows + _NS * 128 - 1) // (_NS * 128)) * (_NS * 128)
    return functools.partial(
        pl.kernel,
        functools.partial(_p2_body, nch, nrows_pad, transform),
        out_type=[jax.ShapeDtypeStruct((_NC, nrows_pad, _H), jnp.float32)],
        mesh=_mesh,
        scratch_types=[pltpu.VMEM((4, 128), jnp.int32),
                       pltpu.VMEM((512, _H), jnp.float32),
                       pltpu.VMEM((2 * _H,), jnp.float32),
                       pltpu.VMEM_SHARED((nrows_pad, _H), jnp.float32),
                       pltpu.SemaphoreType.DMA],
        compiler_params=_sc_params,
    )()


_p2_atom = _make_p2_small(_E, _N, True)
_abf_scatter = _make_p2_small(_E, _N, False)
_p1_line = _make_p1(_LE)


_CHROWS = 20000
_LNPASS = _LN // _CHROWS


def _p2_line_body(nch, p_hbm, d_hbm, out_hbm, ivr, iv2, rv, zbuf, sacc, sem):
    cid = lax.axis_index("c")
    sid = lax.axis_index("s")
    wid = sid * _NC + cid
    base0 = wid * (nch * _CH)
    garbage = _CHROWS
    zt = (_CHROWS + 16) // _NS

    def zrow(r, carry):
        for c in range(4):
            rv[r, pl.ds(c * 16, 16)] = jnp.zeros((16,), jnp.float32)
        return carry
    lax.fori_loop(0, 512, zrow, 0)

    def zrow2(r, carry):
        for c in range(4):
            zbuf[r, pl.ds(c * 16, 16)] = jnp.zeros((16,), jnp.float32)
        return carry
    lax.fori_loop(0, 128, zrow2, 0)

    def one_pass(k, carry):
        lo = k * _CHROWS
        for r in range(9):
            pltpu.sync_copy(zbuf, sacc.at[pl.ds(sid * zt + r * 128, 128)])
        pltpu.sync_copy(zbuf.at[pl.ds(0, zt - 9 * 128)],
                        sacc.at[pl.ds(sid * zt + 9 * 128, zt - 9 * 128)])
        plsc.subcore_barrier()

        def chunk(i, c2):
            base = base0 + i * _CH
            pltpu.sync_copy(d_hbm.at[pl.ds(base, _CH)], ivr)
            pltpu.sync_copy(p_hbm.at[pl.ds(base, _CH)], rv.at[pl.ds(0, _CH)])
            for j in range(25):
                v = ivr[pl.ds(j * 16, 16)]
                adj = v - lo
                ok = (adj >= 0) & (adj < _CHROWS)
                idx = jnp.where(ok, adj, jnp.full((16,), garbage, jnp.int32))
                iv2[j // 8, pl.ds((j % 8) * 16, 16)] = idx
            for kk in range(1, 8):
                iv2[3, pl.ds(kk * 16, 16)] = jnp.full((16,), garbage, jnp.int32)
            for g in range(4):
                pltpu.sync_copy(rv.at[pl.ds(g * 128, 128)],
                                sacc.at[iv2.at[g]], add=True)
            return c2

        lax.fori_loop(0, nch, chunk, 0)
        plsc.subcore_barrier()
        pltpu.sync_copy(
            sacc.at[pl.ds(sid * (_CHROWS // _NS), _CHROWS // _NS)],
            out_hbm.at[cid, pl.ds(lo + sid * (_CHROWS // _NS), _CHROWS // _NS)])
        plsc.subcore_barrier()
        return carry

    lax.fori_loop(0, _LNPASS, one_pass, 0)


_p2_line = functools.partial(
    pl.kernel,
    functools.partial(_p2_line_body, _LE // (_NW * _CH)),
    out_type=[jax.ShapeDtypeStruct((_NC, _LN, _H), jnp.float32)],
    mesh=_mesh,
    scratch_types=[pltpu.VMEM((_CH,), jnp.int32),
                   pltpu.VMEM((4, 128), jnp.int32),
                   pltpu.VMEM((512, _H), jnp.float32),
                   pltpu.VMEM((128, _H), jnp.float32),
                   pltpu.VMEM_SHARED((_CHROWS + 16, _H), jnp.float32),
                   pltpu.SemaphoreType.DMA],
    compiler_params=_sc_params,
)()


def _silu_bn_tc_body(t_ref, a_ref, b_ref, o_ref):
    z = t_ref[...] * a_ref[...] + b_ref[...]
    o_ref[...] = z * jax.nn.sigmoid(z)


def _silu_bn_tc(t, a, b):
    ec = t.shape[0]
    blk = 2000
    return pl.pallas_call(
        _silu_bn_tc_body,
        grid=(ec // blk,),
        in_specs=[pl.BlockSpec((blk, _H), lambda i: (i, 0)),
                  pl.BlockSpec((1, _H), lambda i: (0, 0)),
                  pl.BlockSpec((1, _H), lambda i: (0, 0))],
        out_specs=pl.BlockSpec((blk, _H), lambda i: (i, 0)),
        out_shape=jax.ShapeDtypeStruct((ec, _H), jnp.float32),
    )(t, a[None, :], b[None, :])


def _silu(x):
    return x * jax.nn.sigmoid(x)


def _bn(x, g, be):
    mu = jnp.mean(x, axis=0)
    var = jnp.var(x, axis=0)
    return (x - mu) / jnp.sqrt(var + 1e-5) * g + be


def _seg_mean(v, idx, n):
    s = jax.ops.segment_sum(v, idx, num_segments=n)
    c = jax.ops.segment_sum(jnp.ones((idx.shape[0],), v.dtype), idx, num_segments=n)
    return s / jnp.maximum(c, 1.0)[:, None]


def _bn_consts(st, ec, g, be):
    tot = jnp.sum(st, axis=0)
    mu = tot[:_H] / ec
    var = tot[_H:] / ec - mu * mu
    inv = g / jnp.sqrt(var + 1e-5)
    return inv, be - mu * inv


def _conv_atom(x, dst, src, ea, W1, b1, g1, be1, W2, b2, uW, ub, ug, ube,
               indeg):
    A = x @ W1[:, :_H].T + b1[None, :]
    B = x @ W1[:, _H:2 * _H].T
    w1c = W1[:, 2 * _H]
    t, st = _p1_atom(A, B, dst, src, ea, w1c)
    a_c, b_c = _bn_consts(st, _E, g1, be1)
    sp = _p2_atom(t, dst, jnp.concatenate([a_c, b_c]))[0]
    s = (sp[0] + sp[1])[:_N]
    cnt = jnp.maximum(indeg, 1.0)
    nz = jnp.minimum(indeg, 1.0)
    agg = (s / cnt[:, None]) @ W2.T + nz[:, None] * b2[None, :]
    h = x @ uW[:, :_H].T + agg @ uW[:, _H:].T + ub[None, :]
    h = _silu(_bn(h, ug, ube))
    return h + x


def _conv_line(x, dst, src, ea, W1, b1, g1, be1, W2, b2, uW, ub, ug, ube,
               indeg):
    A = x @ W1[:, :_H].T + b1[None, :]
    B = x @ W1[:, _H:2 * _H].T
    w1c = W1[:, 2 * _H]
    t, st = _p1_line(A, B, dst, src, ea, w1c)
    a_c, b_c = _bn_consts(st, _LE, g1, be1)
    p = _silu_bn_tc(t, a_c, b_c)
    sp = _p2_line(p, dst)[0]
    s = sp[0] + sp[1]
    cnt = jnp.maximum(indeg, 1.0)
    nz = jnp.minimum(indeg, 1.0)
    agg = (s / cnt[:, None]) @ W2.T + nz[:, None] * b2[None, :]
    h = x @ uW[:, :_H].T + agg @ uW[:, _H:].T + ub[None, :]
    h = _silu(_bn(h, ug, ube))
    return h + x


def _out_kernel(g_ref, w_ref, b_ref, o_ref):
    t = jnp.dot(g_ref[...], w_ref[...], preferred_element_type=jnp.float32)
    t = t + b_ref[...]
    o_ref[...] = t * jax.nn.sigmoid(t)


def kernel(x, edge_attr, line_graph_x, line_graph_edge_attr, W_emb_atom, b_emb_atom,
           a_msg_W1, a_msg_b1, a_msg_g, a_msg_be, a_msg_W2, a_msg_b2,
           a_upd_W, a_upd_b, a_upd_g, a_upd_be,
           W_emb_line, b_emb_line, l_msg_W1, l_msg_b1, l_msg_g, l_msg_be,
           l_msg_W2, l_msg_b2, l_upd_W, l_upd_b, l_upd_g, l_upd_be,
           b2a_W, b2a_b, b2a_g, b2a_be, out_W, out_b,
           edge_index, batch, line_graph_edge_index, line_graph_batch_mapping):
    n = x.shape[0]
    h = x @ W_emb_atom.T + b_emb_atom
    lx = line_graph_x @ W_emb_line.T + b_emb_line
    src = edge_index[0]
    dst = edge_index[1]
    ea = edge_attr[:, 0]
    l_src = line_graph_edge_index[0]
    l_dst = line_graph_edge_index[1]
    lea = line_graph_edge_attr[:, 0]
    ones_e = jnp.ones((_E,), jnp.float32)
    bc = jnp.maximum(jax.ops.segment_sum(ones_e, src, num_segments=n), 1.0)
    a_indeg = jax.ops.segment_sum(ones_e, dst, num_segments=n)
    l_indeg = jax.ops.segment_sum(jnp.ones((_LE,), jnp.float32),
                                  line_graph_edge_index[1], num_segments=_LN)

    for i in range(_NJ):
        h = _conv_atom(h, dst, src, ea, a_msg_W1[i], a_msg_b1[i], a_msg_g[i],
                       a_msg_be[i], a_msg_W2[i], a_msg_b2[i], a_upd_W[i],
                       a_upd_b[i], a_upd_g[i], a_upd_be[i], a_indeg)
        lx = _conv_line(lx, l_dst, l_src, lea,
                        l_msg_W1[i], l_msg_b1[i], l_msg_g[i], l_msg_be[i],
                        l_msg_W2[i], l_msg_b2[i], l_upd_W[i], l_upd_b[i],
                        l_upd_g[i], l_upd_be[i], l_indeg)
        abfp = _abf_scatter(lx, src, jnp.zeros((2 * _H,), jnp.float32))[0]
        abf = (abfp[0] + abfp[1])[:_N] / bc[:, None]
        hb = jnp.concatenate([h, abf], axis=1) @ b2a_W[i].T + b2a_b[i]
        h = _silu(_bn(hb, b2a_g[i], b2a_be[i]))
    for i in range(_NJ, _ANL):
        h = _conv_atom(h, dst, src, ea, a_msg_W1[i], a_msg_b1[i], a_msg_g[i],
                       a_msg_be[i], a_msg_W2[i], a_msg_b2[i], a_upd_W[i],
                       a_upd_b[i], a_upd_g[i], a_upd_be[i], a_indeg)

    atom_emb = _seg_mean(h, batch, _NB)
    line_batch = batch[line_graph_batch_mapping]
    line_emb = _seg_mean(lx, line_batch, _NB)
    g = jnp.concatenate([atom_emb, line_emb], axis=1)

    return pl.pallas_call(
        _out_kernel,
        out_shape=jax.ShapeDtypeStruct((_NB, _H), jnp.float32),
    )(g, out_W.T, out_b[None, :])


# speedup vs baseline: 1.9443x; 1.9443x over previous
import functools

import jax
import jax.numpy as jnp
from jax import lax
from jax.experimental import pallas as pl
from jax.experimental.pallas import tpu as pltpu
from jax.experimental.pallas import tpu_sc as plsc

_N = 10000
_E = 320000
_LN = 320000
_LE = 640000
_H = 64
_NB = 64
_ANL = 5
_NJ = 3

_NC = 2
_NS = 16
_NW = _NC * _NS
_CH = 400

_mesh = plsc.VectorSubcoreMesh(core_axis_name="c", subcore_axis_name="s")
_sc_params = pltpu.CompilerParams(use_tc_tiling_on_sc=False)


def _p1_body(nch, A_hbm, B_hbm, d_hbm, s_hbm, ea_hbm, w_hbm,
             t_hbm, st_hbm, dv, sv, eav, wv, Ar, Br, stv, sem):
    wid = lax.axis_index("s") * _NC + lax.axis_index("c")
    base0 = wid * (nch * _CH)
    pltpu.sync_copy(w_hbm, wv)
    for j in range(8):
        stv[pl.ds(j * 16, 16)] = jnp.zeros((16,), jnp.float32)

    def chunk(i, carry):
        base = base0 + i * _CH
        pltpu.sync_copy(d_hbm.at[pl.ds(base, _CH)], dv)
        pltpu.sync_copy(s_hbm.at[pl.ds(base, _CH)], sv)
        pltpu.sync_copy(ea_hbm.at[pl.ds(base, _CH)], eav)
        pltpu.async_copy(A_hbm.at[dv], Ar, sem).wait()
        pltpu.async_copy(B_hbm.at[sv], Br, sem).wait()

        w0 = wv[pl.ds(0, 16)]
        w1 = wv[pl.ds(16, 16)]
        w2 = wv[pl.ds(32, 16)]
        w3 = wv[pl.ds(48, 16)]
        ws = (w0, w1, w2, w3)

        def grp(gi, acc):
            g16 = eav[pl.ds(gi * 16, 16)]
            out = list(acc)
            for j in range(16):
                e = gi * 16 + j
                eab = jnp.full((16,), g16[j])
                for c in range(4):
                    a = Ar[e, pl.ds(c * 16, 16)]
                    b = Br[e, pl.ds(c * 16, 16)]
                    t = a + b + eab * ws[c]
                    Ar[e, pl.ds(c * 16, 16)] = t
                    out[c] = out[c] + t
                    out[4 + c] = out[4 + c] + t * t
            return tuple(out)

        z = jnp.zeros((16,), jnp.float32)
        acc = lax.fori_loop(0, _CH // 16, grp, (z,) * 8)
        for c in range(8):
            stv[pl.ds(c * 16, 16)] = stv[pl.ds(c * 16, 16)] + acc[c]
        pltpu.sync_copy(Ar, t_hbm.at[pl.ds(base, _CH)])
        return carry

    lax.fori_loop(0, nch, chunk, 0)
    pltpu.sync_copy(stv, st_hbm.at[wid])


def _make_p1(ec):
    nch = ec // (_NW * _CH)
    return functools.partial(
        pl.kernel,
        functools.partial(_p1_body, nch),
        out_type=[jax.ShapeDtypeStruct((ec, _H), jnp.float32),
                  jax.ShapeDtypeStruct((_NW, 2 * _H), jnp.float32)],
        mesh=_mesh,
        scratch_types=[pltpu.VMEM((_CH,), jnp.int32),
                       pltpu.VMEM((_CH,), jnp.int32),
                       pltpu.VMEM((_CH,), jnp.float32),
                       pltpu.VMEM((_H,), jnp.float32),
                       pltpu.VMEM((_CH, _H), jnp.float32),
                       pltpu.VMEM((_CH, _H), jnp.float32),
                       pltpu.VMEM((2 * _H,), jnp.float32),
                       pltpu.SemaphoreType.DMA],
        compiler_params=_sc_params,
    )()


_p1_atom = _make_p1(_E)


def _p2_body(nch, nrows_pad, transform, rows_hbm, i_hbm, ab_hbm,
             out_hbm, iv2, rv, abv, sacc, sem):
    cid = lax.axis_index("c")
    sid = lax.axis_index("s")
    wid = sid * _NC + cid
    base0 = wid * (nch * _CH)
    garbage = nrows_pad - 1
    zrows = nrows_pad // _NS

    def zrow(r, carry):
        for c in range(4):
            rv[r, pl.ds(c * 16, 16)] = jnp.zeros((16,), jnp.float32)
        return carry
    lax.fori_loop(0, 512, zrow, 0)

    nfull = zrows // 512
    for r in range(nfull):
        pltpu.sync_copy(rv, sacc.at[pl.ds(sid * zrows + r * 512, 512)])
    rem = zrows - nfull * 512
    if rem:
        pltpu.sync_copy(rv.at[pl.ds(0, rem)],
                        sacc.at[pl.ds(sid * zrows + nfull * 512, rem)])
    plsc.subcore_barrier()

    if transform:
        pltpu.sync_copy(ab_hbm, abv)

    def chunk(i, carry):
        base = base0 + i * _CH
        for g in range(3):
            pltpu.sync_copy(i_hbm.at[pl.ds(base + g * 128, 128)], iv2.at[g])
        pltpu.sync_copy(i_hbm.at[pl.ds(base + 384, 16)],
                        iv2.at[3, pl.ds(0, 16)])
        for k in range(1, 8):
            iv2[3, pl.ds(k * 16, 16)] = jnp.full((16,), garbage, jnp.int32)
        pltpu.sync_copy(rows_hbm.at[pl.ds(base, _CH)], rv.at[pl.ds(0, _CH)])
        if transform:
            def row(e, rc):
                for c in range(4):
                    t = rv[e, pl.ds(c * 16, 16)]
                    z = t * abv[pl.ds(c * 16, 16)] + abv[pl.ds(64 + c * 16, 16)]
                    p = z / (1.0 + jnp.exp(-z))
                    rv[e, pl.ds(c * 16, 16)] = p
                return rc
            lax.fori_loop(0, _CH, row, 0)
        for g in range(4):
            pltpu.sync_copy(rv.at[pl.ds(g * 128, 128)],
                            sacc.at[iv2.at[g]], add=True)
        return carry

    lax.fori_loop(0, nch, chunk, 0)
    plsc.subcore_barrier()
    pltpu.sync_copy(sacc.at[pl.ds(sid * zrows, zrows)],
                    out_hbm.at[cid, pl.ds(sid * zrows, zrows)])


def _make_p2_small(ec, nrows, transform):
    nch = ec // (_NW * _CH)
    nrows_pad = ((nrows + _NS * 128 - 1) // (_NS * 128)) * (_NS * 128)
    return functools.partial(
        pl.kernel,
        functools.partial(_p2_body, nch, nrows_pad, transform),
        out_type=[jax.ShapeDtypeStruct((_NC, nrows_pad, _H), jnp.float32)],
        mesh=_mesh,
        scratch_types=[pltpu.VMEM((4, 128), jnp.int32),
                       pltpu.VMEM((512, _H), jnp.float32),
                       pltpu.VMEM((2 * _H,), jnp.float32),
                       pltpu.VMEM_SHARED((nrows_pad, _H), jnp.float32),
                       pltpu.SemaphoreType.DMA],
        compiler_params=_sc_params,
    )()


_p2_atom = _make_p2_small(_E, _N, True)
_abf_scatter = _make_p2_small(_E, _N, False)
_p1_line = _make_p1(_LE)


_CHROWS = 20000
_LNPASS = _LN // _CHROWS


def _p2_line_body(nch, p_hbm, d_hbm, out_hbm, ivr, iv2, rv, zbuf, sacc, sem):
    cid = lax.axis_index("c")
    sid = lax.axis_index("s")
    wid = sid * _NC + cid
    base0 = wid * (nch * _CH)
    garbage = _CHROWS
    zt = (_CHROWS + 16) // _NS

    def zrow(r, carry):
        for c in range(4):
            rv[r, pl.ds(c * 16, 16)] = jnp.zeros((16,), jnp.float32)
        return carry
    lax.fori_loop(0, 512, zrow, 0)

    def zrow2(r, carry):
        for c in range(4):
            zbuf[r, pl.ds(c * 16, 16)] = jnp.zeros((16,), jnp.float32)
        return carry
    lax.fori_loop(0, 128, zrow2, 0)

    def one_pass(k, carry):
        lo = k * _CHROWS
        for r in range(9):
            pltpu.sync_copy(zbuf, sacc.at[pl.ds(sid * zt + r * 128, 128)])
        pltpu.sync_copy(zbuf.at[pl.ds(0, zt - 9 * 128)],
                        sacc.at[pl.ds(sid * zt + 9 * 128, zt - 9 * 128)])
        plsc.subcore_barrier()

        def chunk(i, c2):
            base = base0 + i * _CH
            pltpu.sync_copy(d_hbm.at[pl.ds(base, _CH)], ivr)
            pltpu.sync_copy(p_hbm.at[pl.ds(base, _CH)], rv.at[pl.ds(0, _CH)])
            for j in range(25):
                v = ivr[pl.ds(j * 16, 16)]
                adj = v - lo
                ok = (adj >= 0) & (adj < _CHROWS)
                idx = jnp.where(ok, adj, jnp.full((16,), garbage, jnp.int32))
                iv2[j // 8, pl.ds((j % 8) * 16, 16)] = idx
            for kk in range(1, 8):
                iv2[3, pl.ds(kk * 16, 16)] = jnp.full((16,), garbage, jnp.int32)
            for g in range(4):
                pltpu.sync_copy(rv.at[pl.ds(g * 128, 128)],
                                sacc.at[iv2.at[g]], add=True)
            return c2

        lax.fori_loop(0, nch, chunk, 0)
        plsc.subcore_barrier()
        pltpu.sync_copy(
            sacc.at[pl.ds(sid * (_CHROWS // _NS), _CHROWS // _NS)],
            out_hbm.at[cid, pl.ds(lo + sid * (_CHROWS // _NS), _CHROWS // _NS)])
        plsc.subcore_barrier()
        return carry

    lax.fori_loop(0, _LNPASS, one_pass, 0)


_p2_line = functools.partial(
    pl.kernel,
    functools.partial(_p2_line_body, _LE // (_NW * _CH)),
    out_type=[jax.ShapeDtypeStruct((_NC, _LN, _H), jnp.float32)],
    mesh=_mesh,
    scratch_types=[pltpu.VMEM((_CH,), jnp.int32),
                   pltpu.VMEM((4, 128), jnp.int32),
                   pltpu.VMEM((512, _H), jnp.float32),
                   pltpu.VMEM((128, _H), jnp.float32),
                   pltpu.VMEM_SHARED((_CHROWS + 16, _H), jnp.float32),
                   pltpu.SemaphoreType.DMA],
    compiler_params=_sc_params,
)()


def _silu_bn_tc_body(t_ref, a_ref, b_ref, o_ref):
    z = t_ref[...] * a_ref[...] + b_ref[...]
    o_ref[...] = z * jax.nn.sigmoid(z)


def _silu_bn_tc(t, a, b):
    ec = t.shape[0]
    blk = 2000
    return pl.pallas_call(
        _silu_bn_tc_body,
        grid=(ec // blk,),
        in_specs=[pl.BlockSpec((blk, _H), lambda i: (i, 0)),
                  pl.BlockSpec((1, _H), lambda i: (0, 0)),
                  pl.BlockSpec((1, _H), lambda i: (0, 0))],
        out_specs=pl.BlockSpec((blk, _H), lambda i: (i, 0)),
        out_shape=jax.ShapeDtypeStruct((ec, _H), jnp.float32),
    )(t, a[None, :], b[None, :])


def _silu(x):
    return x * jax.nn.sigmoid(x)


def _bn(x, g, be):
    mu = jnp.mean(x, axis=0)
    var = jnp.var(x, axis=0)
    return (x - mu) / jnp.sqrt(var + 1e-5) * g + be


def _seg_mean(v, idx, n):
    s = jax.ops.segment_sum(v, idx, num_segments=n)
    c = jax.ops.segment_sum(jnp.ones((idx.shape[0],), v.dtype), idx, num_segments=n)
    return s / jnp.maximum(c, 1.0)[:, None]


def _bn_consts(st, ec, g, be):
    tot = jnp.sum(st, axis=0)
    mu = tot[:_H] / ec
    var = tot[_H:] / ec - mu * mu
    inv = g / jnp.sqrt(var + 1e-5)
    return inv, be - mu * inv


def _conv_atom(x, dst, src, ea, W1, b1, g1, be1, W2, b2, uW, ub, ug, ube,
               indeg):
    A = x @ W1[:, :_H].T + b1[None, :]
    B = x @ W1[:, _H:2 * _H].T
    w1c = W1[:, 2 * _H]
    t, st = _p1_atom(A, B, dst, src, ea, w1c)
    a_c, b_c = _bn_consts(st, _E, g1, be1)
    sp = _p2_atom(t, dst, jnp.concatenate([a_c, b_c]))[0]
    s = (sp[0] + sp[1])[:_N]
    cnt = jnp.maximum(indeg, 1.0)
    nz = jnp.minimum(indeg, 1.0)
    agg = (s / cnt[:, None]) @ W2.T + nz[:, None] * b2[None, :]
    h = x @ uW[:, :_H].T + agg @ uW[:, _H:].T + ub[None, :]
    h = _silu(_bn(h, ug, ube))
    return h + x


def _conv_line(x, dst, src, ea, W1, b1, g1, be1, W2, b2, uW, ub, ug, ube,
               indeg):
    A = x @ W1[:, :_H].T + b1[None, :]
    B = x @ W1[:, _H:2 * _H].T
    w1c = W1[:, 2 * _H]
    t, st = _p1_line(A, B, dst, src, ea, w1c)
    a_c, b_c = _bn_consts(st, _LE, g1, be1)
    p = _silu_bn_tc(t, a_c, b_c)
    s = jax.ops.segment_sum(p, dst, num_segments=_LN)
    cnt = jnp.maximum(indeg, 1.0)
    nz = jnp.minimum(indeg, 1.0)
    agg = (s / cnt[:, None]) @ W2.T + nz[:, None] * b2[None, :]
    h = x @ uW[:, :_H].T + agg @ uW[:, _H:].T + ub[None, :]
    h = _silu(_bn(h, ug, ube))
    return h + x


def _out_kernel(g_ref, w_ref, b_ref, o_ref):
    t = jnp.dot(g_ref[...], w_ref[...], preferred_element_type=jnp.float32)
    t = t + b_ref[...]
    o_ref[...] = t * jax.nn.sigmoid(t)


def kernel(x, edge_attr, line_graph_x, line_graph_edge_attr, W_emb_atom, b_emb_atom,
           a_msg_W1, a_msg_b1, a_msg_g, a_msg_be, a_msg_W2, a_msg_b2,
           a_upd_W, a_upd_b, a_upd_g, a_upd_be,
           W_emb_line, b_emb_line, l_msg_W1, l_msg_b1, l_msg_g, l_msg_be,
           l_msg_W2, l_msg_b2, l_upd_W, l_upd_b, l_upd_g, l_upd_be,
           b2a_W, b2a_b, b2a_g, b2a_be, out_W, out_b,
           edge_index, batch, line_graph_edge_index, line_graph_batch_mapping):
    n = x.shape[0]
    h = x @ W_emb_atom.T + b_emb_atom
    lx = line_graph_x @ W_emb_line.T + b_emb_line
    src = edge_index[0]
    dst = edge_index[1]
    ea = edge_attr[:, 0]
    l_src = line_graph_edge_index[0]
    l_dst = line_graph_edge_index[1]
    lea = line_graph_edge_attr[:, 0]
    ones_e = jnp.ones((_E,), jnp.float32)
    bc = jnp.maximum(jax.ops.segment_sum(ones_e, src, num_segments=n), 1.0)
    a_indeg = jax.ops.segment_sum(ones_e, dst, num_segments=n)
    l_indeg = jax.ops.segment_sum(jnp.ones((_LE,), jnp.float32),
                                  line_graph_edge_index[1], num_segments=_LN)

    for i in range(_NJ):
        h = _conv_atom(h, dst, src, ea, a_msg_W1[i], a_msg_b1[i], a_msg_g[i],
                       a_msg_be[i], a_msg_W2[i], a_msg_b2[i], a_upd_W[i],
                       a_upd_b[i], a_upd_g[i], a_upd_be[i], a_indeg)
        lx = _conv_line(lx, l_dst, l_src, lea,
                        l_msg_W1[i], l_msg_b1[i], l_msg_g[i], l_msg_be[i],
                        l_msg_W2[i], l_msg_b2[i], l_upd_W[i], l_upd_b[i],
                        l_upd_g[i], l_upd_be[i], l_indeg)
        abfp = _abf_scatter(lx, src, jnp.zeros((2 * _H,), jnp.float32))[0]
        abf = (abfp[0] + abfp[1])[:_N] / bc[:, None]
        hb = jnp.concatenate([h, abf], axis=1) @ b2a_W[i].T + b2a_b[i]
        h = _silu(_bn(hb, b2a_g[i], b2a_be[i]))
    for i in range(_NJ, _ANL):
        h = _conv_atom(h, dst, src, ea, a_msg_W1[i], a_msg_b1[i], a_msg_g[i],
                       a_msg_be[i], a_msg_W2[i], a_msg_b2[i], a_upd_W[i],
                       a_upd_b[i], a_upd_g[i], a_upd_be[i], a_indeg)

    atom_emb = _seg_mean(h, batch, _NB)
    line_batch = batch[line_graph_batch_mapping]
    line_emb = _seg_mean(lx, line_batch, _NB)
    g = jnp.concatenate([atom_emb, line_emb], axis=1)

    return pl.pallas_call(
        _out_kernel,
        out_shape=jax.ShapeDtypeStruct((_NB, _H), jnp.float32),
    )(g, out_W.T, out_b[None, :])


# speedup vs baseline: 2.0037x; 1.0305x over previous
import functools

import jax
import jax.numpy as jnp
from jax import lax
from jax.experimental import pallas as pl
from jax.experimental.pallas import tpu as pltpu
from jax.experimental.pallas import tpu_sc as plsc

_N = 10000
_E = 320000
_LN = 320000
_LE = 640000
_H = 64
_NB = 64
_ANL = 5
_NJ = 3

_NC = 2
_NS = 16
_NW = _NC * _NS
_CH = 400

_mesh = plsc.VectorSubcoreMesh(core_axis_name="c", subcore_axis_name="s")
_sc_params = pltpu.CompilerParams(use_tc_tiling_on_sc=False)


def _p1_body(nch, A_hbm, B_hbm, d_hbm, s_hbm, ea_hbm, w_hbm,
             t_hbm, st_hbm, dv, sv, eav, wv, Ar, Br, stv, sem):
    wid = lax.axis_index("s") * _NC + lax.axis_index("c")
    base0 = wid * (nch * _CH)
    pltpu.sync_copy(w_hbm, wv)
    for j in range(8):
        stv[pl.ds(j * 16, 16)] = jnp.zeros((16,), jnp.float32)

    def chunk(i, carry):
        base = base0 + i * _CH
        c1 = pltpu.async_copy(d_hbm.at[pl.ds(base, _CH)], dv, sem)
        c2 = pltpu.async_copy(s_hbm.at[pl.ds(base, _CH)], sv, sem)
        c3 = pltpu.async_copy(ea_hbm.at[pl.ds(base, _CH)], eav, sem)
        c1.wait(); c2.wait(); c3.wait()
        g1 = pltpu.async_copy(A_hbm.at[dv], Ar, sem)
        g2 = pltpu.async_copy(B_hbm.at[sv], Br, sem)
        g1.wait(); g2.wait()

        w0 = wv[pl.ds(0, 16)]
        w1 = wv[pl.ds(16, 16)]
        w2 = wv[pl.ds(32, 16)]
        w3 = wv[pl.ds(48, 16)]
        ws = (w0, w1, w2, w3)

        def grp(gi, acc):
            g16 = eav[pl.ds(gi * 16, 16)]
            out = list(acc)
            for j in range(16):
                e = gi * 16 + j
                eab = jnp.full((16,), g16[j])
                for c in range(4):
                    a = Ar[e, pl.ds(c * 16, 16)]
                    b = Br[e, pl.ds(c * 16, 16)]
                    t = a + b + eab * ws[c]
                    Ar[e, pl.ds(c * 16, 16)] = t
                    out[c] = out[c] + t
                    out[4 + c] = out[4 + c] + t * t
            return tuple(out)

        z = jnp.zeros((16,), jnp.float32)
        acc = lax.fori_loop(0, _CH // 16, grp, (z,) * 8)
        for c in range(8):
            stv[pl.ds(c * 16, 16)] = stv[pl.ds(c * 16, 16)] + acc[c]
        pltpu.sync_copy(Ar, t_hbm.at[pl.ds(base, _CH)])
        return carry

    lax.fori_loop(0, nch, chunk, 0)
    pltpu.sync_copy(stv, st_hbm.at[wid])


def _make_p1(ec):
    nch = ec // (_NW * _CH)
    return functools.partial(
        pl.kernel,
        functools.partial(_p1_body, nch),
        out_type=[jax.ShapeDtypeStruct((ec, _H), jnp.float32),
                  jax.ShapeDtypeStruct((_NW, 2 * _H), jnp.float32)],
        mesh=_mesh,
        scratch_types=[pltpu.VMEM((_CH,), jnp.int32),
                       pltpu.VMEM((_CH,), jnp.int32),
                       pltpu.VMEM((_CH,), jnp.float32),
                       pltpu.VMEM((_H,), jnp.float32),
                       pltpu.VMEM((_CH, _H), jnp.float32),
                       pltpu.VMEM((_CH, _H), jnp.float32),
                       pltpu.VMEM((2 * _H,), jnp.float32),
                       pltpu.SemaphoreType.DMA],
        compiler_params=_sc_params,
    )()


_p1_atom = _make_p1(_E)


def _p2_body(nch, nrows_pad, transform, rows_hbm, i_hbm, ab_hbm,
             out_hbm, iv2, rv, abv, sacc, sem):
    cid = lax.axis_index("c")
    sid = lax.axis_index("s")
    wid = sid * _NC + cid
    base0 = wid * (nch * _CH)
    garbage = nrows_pad - 1
    zrows = nrows_pad // _NS

    def zrow(r, carry):
        for c in range(4):
            rv[r, pl.ds(c * 16, 16)] = jnp.zeros((16,), jnp.float32)
        return carry
    lax.fori_loop(0, 512, zrow, 0)

    nfull = zrows // 512
    for r in range(nfull):
        pltpu.sync_copy(rv, sacc.at[pl.ds(sid * zrows + r * 512, 512)])
    rem = zrows - nfull * 512
    if rem:
        pltpu.sync_copy(rv.at[pl.ds(0, rem)],
                        sacc.at[pl.ds(sid * zrows + nfull * 512, rem)])
    plsc.subcore_barrier()

    if transform:
        pltpu.sync_copy(ab_hbm, abv)

    def chunk(i, carry):
        base = base0 + i * _CH
        cs = [pltpu.async_copy(i_hbm.at[pl.ds(base + g * 128, 128)],
                               iv2.at[g], sem) for g in range(3)]
        cs.append(pltpu.async_copy(i_hbm.at[pl.ds(base + 384, 16)],
                                   iv2.at[3, pl.ds(0, 16)], sem))
        cs.append(pltpu.async_copy(rows_hbm.at[pl.ds(base, _CH)],
                                   rv.at[pl.ds(0, _CH)], sem))
        for c in cs:
            c.wait()
        for k in range(1, 8):
            iv2[3, pl.ds(k * 16, 16)] = jnp.full((16,), garbage, jnp.int32)
        if transform:
            def row(e, rc):
                for c in range(4):
                    t = rv[e, pl.ds(c * 16, 16)]
                    z = t * abv[pl.ds(c * 16, 16)] + abv[pl.ds(64 + c * 16, 16)]
                    p = z / (1.0 + jnp.exp(-z))
                    rv[e, pl.ds(c * 16, 16)] = p
                return rc
            lax.fori_loop(0, _CH, row, 0)
        ws_ = [pltpu.async_copy(rv.at[pl.ds(g * 128, 128)],
                                sacc.at[iv2.at[g]], sem, add=True)
               for g in range(4)]
        for w_ in ws_:
            w_.wait()
        return carry

    lax.fori_loop(0, nch, chunk, 0)
    plsc.subcore_barrier()
    pltpu.sync_copy(sacc.at[pl.ds(sid * zrows, zrows)],
                    out_hbm.at[cid, pl.ds(sid * zrows, zrows)])


def _make_p2_small(ec, nrows, transform):
    nch = ec // (_NW * _CH)
    nrows_pad = ((nrows + _NS * 128 - 1) // (_NS * 128)) * (_NS * 128)
    return functools.partial(
        pl.kernel,
        functools.partial(_p2_body, nch, nrows_pad, transform),
        out_type=[jax.ShapeDtypeStruct((_NC, nrows_pad, _H), jnp.float32)],
        mesh=_mesh,
        scratch_types=[pltpu.VMEM((4, 128), jnp.int32),
                       pltpu.VMEM((512, _H), jnp.float32),
                       pltpu.VMEM((2 * _H,), jnp.float32),
                       pltpu.VMEM_SHARED((nrows_pad, _H), jnp.float32),
                       pltpu.SemaphoreType.DMA],
        compiler_params=_sc_params,
    )()


_p2_atom = _make_p2_small(_E, _N, True)
_abf_scatter = _make_p2_small(_E, _N, False)
_p1_line = _make_p1(_LE)


_CHROWS = 20000
_LNPASS = _LN // _CHROWS


def _p2_line_body(nch, p_hbm, d_hbm, out_hbm, ivr, iv2, rv, zbuf, sacc, sem):
    cid = lax.axis_index("c")
    sid = lax.axis_index("s")
    wid = sid * _NC + cid
    base0 = wid * (nch * _CH)
    garbage = _CHROWS
    zt = (_CHROWS + 16) // _NS

    def zrow(r, carry):
        for c in range(4):
            rv[r, pl.ds(c * 16, 16)] = jnp.zeros((16,), jnp.float32)
        return carry
    lax.fori_loop(0, 512, zrow, 0)

    def zrow2(r, carry):
        for c in range(4):
            zbuf[r, pl.ds(c * 16, 16)] = jnp.zeros((16,), jnp.float32)
        return carry
    lax.fori_loop(0, 128, zrow2, 0)

    def one_pass(k, carry):
        lo = k * _CHROWS
        for r in range(9):
            pltpu.sync_copy(zbuf, sacc.at[pl.ds(sid * zt + r * 128, 128)])
        pltpu.sync_copy(zbuf.at[pl.ds(0, zt - 9 * 128)],
                        sacc.at[pl.ds(sid * zt + 9 * 128, zt - 9 * 128)])
        plsc.subcore_barrier()

        def chunk(i, c2):
            base = base0 + i * _CH
            pltpu.sync_copy(d_hbm.at[pl.ds(base, _CH)], ivr)
            pltpu.sync_copy(p_hbm.at[pl.ds(base, _CH)], rv.at[pl.ds(0, _CH)])
            for j in range(25):
                v = ivr[pl.ds(j * 16, 16)]
                adj = v - lo
                ok = (adj >= 0) & (adj < _CHROWS)
                idx = jnp.where(ok, adj, jnp.full((16,), garbage, jnp.int32))
                iv2[j // 8, pl.ds((j % 8) * 16, 16)] = idx
            for kk in range(1, 8):
                iv2[3, pl.ds(kk * 16, 16)] = jnp.full((16,), garbage, jnp.int32)
            for g in range(4):
                pltpu.sync_copy(rv.at[pl.ds(g * 128, 128)],
                                sacc.at[iv2.at[g]], add=True)
            return c2

        lax.fori_loop(0, nch, chunk, 0)
        plsc.subcore_barrier()
        pltpu.sync_copy(
            sacc.at[pl.ds(sid * (_CHROWS // _NS), _CHROWS // _NS)],
            out_hbm.at[cid, pl.ds(lo + sid * (_CHROWS // _NS), _CHROWS // _NS)])
        plsc.subcore_barrier()
        return carry

    lax.fori_loop(0, _LNPASS, one_pass, 0)


_p2_line = functools.partial(
    pl.kernel,
    functools.partial(_p2_line_body, _LE // (_NW * _CH)),
    out_type=[jax.ShapeDtypeStruct((_NC, _LN, _H), jnp.float32)],
    mesh=_mesh,
    scratch_types=[pltpu.VMEM((_CH,), jnp.int32),
                   pltpu.VMEM((4, 128), jnp.int32),
                   pltpu.VMEM((512, _H), jnp.float32),
                   pltpu.VMEM((128, _H), jnp.float32),
                   pltpu.VMEM_SHARED((_CHROWS + 16, _H), jnp.float32),
                   pltpu.SemaphoreType.DMA],
    compiler_params=_sc_params,
)()


def _silu_bn_tc_body(t_ref, a_ref, b_ref, o_ref):
    z = t_ref[...] * a_ref[...] + b_ref[...]
    o_ref[...] = z * jax.nn.sigmoid(z)


def _silu_bn_tc(t, a, b):
    ec = t.shape[0]
    blk = 2000
    return pl.pallas_call(
        _silu_bn_tc_body,
        grid=(ec // blk,),
        in_specs=[pl.BlockSpec((blk, _H), lambda i: (i, 0)),
                  pl.BlockSpec((1, _H), lambda i: (0, 0)),
                  pl.BlockSpec((1, _H), lambda i: (0, 0))],
        out_specs=pl.BlockSpec((blk, _H), lambda i: (i, 0)),
        out_shape=jax.ShapeDtypeStruct((ec, _H), jnp.float32),
    )(t, a[None, :], b[None, :])


def _silu(x):
    return x * jax.nn.sigmoid(x)


def _bn(x, g, be):
    mu = jnp.mean(x, axis=0)
    var = jnp.var(x, axis=0)
    return (x - mu) / jnp.sqrt(var + 1e-5) * g + be


def _seg_mean(v, idx, n):
    s = jax.ops.segment_sum(v, idx, num_segments=n)
    c = jax.ops.segment_sum(jnp.ones((idx.shape[0],), v.dtype), idx, num_segments=n)
    return s / jnp.maximum(c, 1.0)[:, None]


def _bn_consts(st, ec, g, be):
    tot = jnp.sum(st, axis=0)
    mu = tot[:_H] / ec
    var = tot[_H:] / ec - mu * mu
    inv = g / jnp.sqrt(var + 1e-5)
    return inv, be - mu * inv


def _conv_atom(x, dst, src, ea, W1, b1, g1, be1, W2, b2, uW, ub, ug, ube,
               indeg):
    A = x @ W1[:, :_H].T + b1[None, :]
    B = x @ W1[:, _H:2 * _H].T
    w1c = W1[:, 2 * _H]
    t, st = _p1_atom(A, B, dst, src, ea, w1c)
    a_c, b_c = _bn_consts(st, _E, g1, be1)
    sp = _p2_atom(t, dst, jnp.concatenate([a_c, b_c]))[0]
    s = (sp[0] + sp[1])[:_N]
    cnt = jnp.maximum(indeg, 1.0)
    nz = jnp.minimum(indeg, 1.0)
    agg = (s / cnt[:, None]) @ W2.T + nz[:, None] * b2[None, :]
    h = x @ uW[:, :_H].T + agg @ uW[:, _H:].T + ub[None, :]
    h = _silu(_bn(h, ug, ube))
    return h + x


def _conv_line(x, dst, src, ea, W1, b1, g1, be1, W2, b2, uW, ub, ug, ube,
               indeg):
    A = x @ W1[:, :_H].T + b1[None, :]
    B = x @ W1[:, _H:2 * _H].T
    w1c = W1[:, 2 * _H]
    t, st = _p1_line(A, B, dst, src, ea, w1c)
    a_c, b_c = _bn_consts(st, _LE, g1, be1)
    p = _silu_bn_tc(t, a_c, b_c)
    s = jax.ops.segment_sum(p, dst, num_segments=_LN)
    cnt = jnp.maximum(indeg, 1.0)
    nz = jnp.minimum(indeg, 1.0)
    agg = (s / cnt[:, None]) @ W2.T + nz[:, None] * b2[None, :]
    h = x @ uW[:, :_H].T + agg @ uW[:, _H:].T + ub[None, :]
    h = _silu(_bn(h, ug, ube))
    return h + x


def _out_kernel(g_ref, w_ref, b_ref, o_ref):
    t = jnp.dot(g_ref[...], w_ref[...], preferred_element_type=jnp.float32)
    t = t + b_ref[...]
    o_ref[...] = t * jax.nn.sigmoid(t)


def kernel(x, edge_attr, line_graph_x, line_graph_edge_attr, W_emb_atom, b_emb_atom,
           a_msg_W1, a_msg_b1, a_msg_g, a_msg_be, a_msg_W2, a_msg_b2,
           a_upd_W, a_upd_b, a_upd_g, a_upd_be,
           W_emb_line, b_emb_line, l_msg_W1, l_msg_b1, l_msg_g, l_msg_be,
           l_msg_W2, l_msg_b2, l_upd_W, l_upd_b, l_upd_g, l_upd_be,
           b2a_W, b2a_b, b2a_g, b2a_be, out_W, out_b,
           edge_index, batch, line_graph_edge_index, line_graph_batch_mapping):
    n = x.shape[0]
    h = x @ W_emb_atom.T + b_emb_atom
    lx = line_graph_x @ W_emb_line.T + b_emb_line
    src = edge_index[0]
    dst = edge_index[1]
    ea = edge_attr[:, 0]
    l_src = line_graph_edge_index[0]
    l_dst = line_graph_edge_index[1]
    lea = line_graph_edge_attr[:, 0]
    ones_e = jnp.ones((_E,), jnp.float32)
    bc = jnp.maximum(jax.ops.segment_sum(ones_e, src, num_segments=n), 1.0)
    a_indeg = jax.ops.segment_sum(ones_e, dst, num_segments=n)
    l_indeg = jax.ops.segment_sum(jnp.ones((_LE,), jnp.float32),
                                  line_graph_edge_index[1], num_segments=_LN)

    for i in range(_NJ):
        h = _conv_atom(h, dst, src, ea, a_msg_W1[i], a_msg_b1[i], a_msg_g[i],
                       a_msg_be[i], a_msg_W2[i], a_msg_b2[i], a_upd_W[i],
                       a_upd_b[i], a_upd_g[i], a_upd_be[i], a_indeg)
        lx = _conv_line(lx, l_dst, l_src, lea,
                        l_msg_W1[i], l_msg_b1[i], l_msg_g[i], l_msg_be[i],
                        l_msg_W2[i], l_msg_b2[i], l_upd_W[i], l_upd_b[i],
                        l_upd_g[i], l_upd_be[i], l_indeg)
        abfp = _abf_scatter(lx, src, jnp.zeros((2 * _H,), jnp.float32))[0]
        abf = (abfp[0] + abfp[1])[:_N] / bc[:, None]
        hb = jnp.concatenate([h, abf], axis=1) @ b2a_W[i].T + b2a_b[i]
        h = _silu(_bn(hb, b2a_g[i], b2a_be[i]))
    for i in range(_NJ, _ANL):
        h = _conv_atom(h, dst, src, ea, a_msg_W1[i], a_msg_b1[i], a_msg_g[i],
                       a_msg_be[i], a_msg_W2[i], a_msg_b2[i], a_upd_W[i],
                       a_upd_b[i], a_upd_g[i], a_upd_be[i], a_indeg)

    atom_emb = _seg_mean(h, batch, _NB)
    line_batch = batch[line_graph_batch_mapping]
    line_emb = _seg_mean(lx, line_batch, _NB)
    g = jnp.concatenate([atom_emb, line_emb], axis=1)

    return pl.pallas_call(
        _out_kernel,
        out_shape=jax.ShapeDtypeStruct((_NB, _H), jnp.float32),
    )(g, out_W.T, out_b[None, :])


# speedup vs baseline: 2.0108x; 1.0035x over previous
import functools

import jax
import jax.numpy as jnp
from jax import lax
from jax.experimental import pallas as pl
from jax.experimental.pallas import tpu as pltpu
from jax.experimental.pallas import tpu_sc as plsc

_N = 10000
_E = 320000
_LN = 320000
_LE = 640000
_H = 64
_NB = 64
_ANL = 5
_NJ = 3

_NC = 2
_NS = 16
_NW = _NC * _NS
_CH = 400

_mesh = plsc.VectorSubcoreMesh(core_axis_name="c", subcore_axis_name="s")
_sc_params = pltpu.CompilerParams(use_tc_tiling_on_sc=False)


def _p1_body(nch, ch, A_hbm, B_hbm, d_hbm, s_hbm, ea_hbm, w_hbm,
             t_hbm, st_hbm, dv, sv, eav, wv, Ar, Br, stv, sem):
    wid = lax.axis_index("s") * _NC + lax.axis_index("c")
    base0 = wid * (nch * ch)
    pltpu.sync_copy(w_hbm, wv)
    for j in range(8):
        stv[pl.ds(j * 16, 16)] = jnp.zeros((16,), jnp.float32)

    def chunk(i, carry):
        base = base0 + i * ch
        c1 = pltpu.async_copy(d_hbm.at[pl.ds(base, ch)], dv, sem)
        c2 = pltpu.async_copy(s_hbm.at[pl.ds(base, ch)], sv, sem)
        c3 = pltpu.async_copy(ea_hbm.at[pl.ds(base, ch)], eav, sem)
        c1.wait(); c2.wait(); c3.wait()
        g1 = pltpu.async_copy(A_hbm.at[dv], Ar, sem)
        g2 = pltpu.async_copy(B_hbm.at[sv], Br, sem)
        g1.wait(); g2.wait()

        w0 = wv[pl.ds(0, 16)]
        w1 = wv[pl.ds(16, 16)]
        w2 = wv[pl.ds(32, 16)]
        w3 = wv[pl.ds(48, 16)]
        ws = (w0, w1, w2, w3)

        def grp(gi, acc):
            g16 = eav[pl.ds(gi * 16, 16)]
            out = list(acc)
            for j in range(16):
                e = gi * 16 + j
                eab = jnp.full((16,), g16[j])
                for c in range(4):
                    a = Ar[e, pl.ds(c * 16, 16)]
                    b = Br[e, pl.ds(c * 16, 16)]
                    t = a + b + eab * ws[c]
                    Ar[e, pl.ds(c * 16, 16)] = t
                    out[c] = out[c] + t
                    out[4 + c] = out[4 + c] + t * t
            return tuple(out)

        z = jnp.zeros((16,), jnp.float32)
        acc = lax.fori_loop(0, ch // 16, grp, (z,) * 8)
        for c in range(8):
            stv[pl.ds(c * 16, 16)] = stv[pl.ds(c * 16, 16)] + acc[c]
        pltpu.sync_copy(Ar, t_hbm.at[pl.ds(base, ch)])
        return carry

    lax.fori_loop(0, nch, chunk, 0)
    pltpu.sync_copy(stv, st_hbm.at[wid])


def _make_p1(ec, ch):
    nch = ec // (_NW * ch)
    return functools.partial(
        pl.kernel,
        functools.partial(_p1_body, nch, ch),
        out_type=[jax.ShapeDtypeStruct((ec, _H), jnp.float32),
                  jax.ShapeDtypeStruct((_NW, 2 * _H), jnp.float32)],
        mesh=_mesh,
        scratch_types=[pltpu.VMEM((ch,), jnp.int32),
                       pltpu.VMEM((ch,), jnp.int32),
                       pltpu.VMEM((ch,), jnp.float32),
                       pltpu.VMEM((_H,), jnp.float32),
                       pltpu.VMEM((ch, _H), jnp.float32),
                       pltpu.VMEM((ch, _H), jnp.float32),
                       pltpu.VMEM((2 * _H,), jnp.float32),
                       pltpu.SemaphoreType.DMA],
        compiler_params=_sc_params,
    )()


_p1_atom = _make_p1(_E, _CH)


def _p2_body(nch, nrows_pad, transform, rows_hbm, i_hbm, ab_hbm,
             out_hbm, iv2, rv, abv, sacc, sem):
    cid = lax.axis_index("c")
    sid = lax.axis_index("s")
    wid = sid * _NC + cid
    base0 = wid * (nch * _CH)
    garbage = nrows_pad - 1
    zrows = nrows_pad // _NS

    def zrow(r, carry):
        for c in range(4):
            rv[r, pl.ds(c * 16, 16)] = jnp.zeros((16,), jnp.float32)
        return carry
    lax.fori_loop(0, 512, zrow, 0)

    nfull = zrows // 512
    for r in range(nfull):
        pltpu.sync_copy(rv, sacc.at[pl.ds(sid * zrows + r * 512, 512)])
    rem = zrows - nfull * 512
    if rem:
        pltpu.sync_copy(rv.at[pl.ds(0, rem)],
                        sacc.at[pl.ds(sid * zrows + nfull * 512, rem)])
    plsc.subcore_barrier()

    if transform:
        pltpu.sync_copy(ab_hbm, abv)

    def chunk(i, carry):
        base = base0 + i * _CH
        cs = [pltpu.async_copy(i_hbm.at[pl.ds(base + g * 128, 128)],
                               iv2.at[g], sem) for g in range(3)]
        cs.append(pltpu.async_copy(i_hbm.at[pl.ds(base + 384, 16)],
                                   iv2.at[3, pl.ds(0, 16)], sem))
        cs.append(pltpu.async_copy(rows_hbm.at[pl.ds(base, _CH)],
                                   rv.at[pl.ds(0, _CH)], sem))
        for c in cs:
            c.wait()
        for k in range(1, 8):
            iv2[3, pl.ds(k * 16, 16)] = jnp.full((16,), garbage, jnp.int32)
        if transform:
            def row(e, rc):
                for c in range(4):
                    t = rv[e, pl.ds(c * 16, 16)]
                    z = t * abv[pl.ds(c * 16, 16)] + abv[pl.ds(64 + c * 16, 16)]
                    p = z / (1.0 + jnp.exp(-z))
                    rv[e, pl.ds(c * 16, 16)] = p
                return rc
            lax.fori_loop(0, _CH, row, 0)
        ws_ = [pltpu.async_copy(rv.at[pl.ds(g * 128, 128)],
                                sacc.at[iv2.at[g]], sem, add=True)
               for g in range(4)]
        for w_ in ws_:
            w_.wait()
        return carry

    lax.fori_loop(0, nch, chunk, 0)
    plsc.subcore_barrier()
    pltpu.sync_copy(sacc.at[pl.ds(sid * zrows, zrows)],
                    out_hbm.at[cid, pl.ds(sid * zrows, zrows)])


def _make_p2_small(ec, nrows, transform):
    nch = ec // (_NW * _CH)
    nrows_pad = ((nrows + _NS * 128 - 1) // (_NS * 128)) * (_NS * 128)
    return functools.partial(
        pl.kernel,
        functools.partial(_p2_body, nch, nrows_pad, transform),
        out_type=[jax.ShapeDtypeStruct((_NC, nrows_pad, _H), jnp.float32)],
        mesh=_mesh,
        scratch_types=[pltpu.VMEM((4, 128), jnp.int32),
                       pltpu.VMEM((512, _H), jnp.float32),
                       pltpu.VMEM((2 * _H,), jnp.float32),
                       pltpu.VMEM_SHARED((nrows_pad, _H), jnp.float32),
                       pltpu.SemaphoreType.DMA],
        compiler_params=_sc_params,
    )()


_p2_atom = _make_p2_small(_E, _N, True)
_abf_scatter = _make_p2_small(_E, _N, False)
_p1_line = _make_p1(_LE, 800)


def _silu_bn_tc_body(t_ref, a_ref, b_ref, o_ref):
    z = t_ref[...] * a_ref[...] + b_ref[...]
    o_ref[...] = z * jax.nn.sigmoid(z)


def _silu_bn_tc(t, a, b):
    ec = t.shape[0]
    blk = 2000
    return pl.pallas_call(
        _silu_bn_tc_body,
        grid=(ec // blk,),
        in_specs=[pl.BlockSpec((blk, _H), lambda i: (i, 0)),
                  pl.BlockSpec((1, _H), lambda i: (0, 0)),
                  pl.BlockSpec((1, _H), lambda i: (0, 0))],
        out_specs=pl.BlockSpec((blk, _H), lambda i: (i, 0)),
        out_shape=jax.ShapeDtypeStruct((ec, _H), jnp.float32),
    )(t, a[None, :], b[None, :])


def _silu(x):
    return x * jax.nn.sigmoid(x)


def _bn(x, g, be):
    mu = jnp.mean(x, axis=0)
    var = jnp.var(x, axis=0)
    return (x - mu) / jnp.sqrt(var + 1e-5) * g + be


def _seg_mean(v, idx, n):
    s = jax.ops.segment_sum(v, idx, num_segments=n)
    c = jax.ops.segment_sum(jnp.ones((idx.shape[0],), v.dtype), idx, num_segments=n)
    return s / jnp.maximum(c, 1.0)[:, None]


def _bn_consts(st, ec, g, be):
    tot = jnp.sum(st, axis=0)
    mu = tot[:_H] / ec
    var = tot[_H:] / ec - mu * mu
    inv = g / jnp.sqrt(var + 1e-5)
    return inv, be - mu * inv


def _conv_atom(x, dst, src, ea, W1, b1, g1, be1, W2, b2, uW, ub, ug, ube,
               indeg):
    A = x @ W1[:, :_H].T + b1[None, :]
    B = x @ W1[:, _H:2 * _H].T
    w1c = W1[:, 2 * _H]
    t, st = _p1_atom(A, B, dst, src, ea, w1c)
    a_c, b_c = _bn_consts(st, _E, g1, be1)
    sp = _p2_atom(t, dst, jnp.concatenate([a_c, b_c]))[0]
    s = (sp[0] + sp[1])[:_N]
    cnt = jnp.maximum(indeg, 1.0)
    nz = jnp.minimum(indeg, 1.0)
    agg = (s / cnt[:, None]) @ W2.T + nz[:, None] * b2[None, :]
    h = x @ uW[:, :_H].T + agg @ uW[:, _H:].T + ub[None, :]
    h = _silu(_bn(h, ug, ube))
    return h + x


def _conv_line(x, dst, src, ea, W1, b1, g1, be1, W2, b2, uW, ub, ug, ube,
               indeg):
    A = x @ W1[:, :_H].T + b1[None, :]
    B = x @ W1[:, _H:2 * _H].T
    w1c = W1[:, 2 * _H]
    t, st = _p1_line(A, B, dst, src, ea, w1c)
    a_c, b_c = _bn_consts(st, _LE, g1, be1)
    p = _silu_bn_tc(t, a_c, b_c)
    s = jax.ops.segment_sum(p, dst, num_segments=_LN)
    cnt = jnp.maximum(indeg, 1.0)
    nz = jnp.minimum(indeg, 1.0)
    agg = (s / cnt[:, None]) @ W2.T + nz[:, None] * b2[None, :]
    h = x @ uW[:, :_H].T + agg @ uW[:, _H:].T + ub[None, :]
    h = _silu(_bn(h, ug, ube))
    return h + x


def _out_kernel(g_ref, w_ref, b_ref, o_ref):
    t = jnp.dot(g_ref[...], w_ref[...], preferred_element_type=jnp.float32)
    t = t + b_ref[...]
    o_ref[...] = t * jax.nn.sigmoid(t)


def kernel(x, edge_attr, line_graph_x, line_graph_edge_attr, W_emb_atom, b_emb_atom,
           a_msg_W1, a_msg_b1, a_msg_g, a_msg_be, a_msg_W2, a_msg_b2,
           a_upd_W, a_upd_b, a_upd_g, a_upd_be,
           W_emb_line, b_emb_line, l_msg_W1, l_msg_b1, l_msg_g, l_msg_be,
           l_msg_W2, l_msg_b2, l_upd_W, l_upd_b, l_upd_g, l_upd_be,
           b2a_W, b2a_b, b2a_g, b2a_be, out_W, out_b,
           edge_index, batch, line_graph_edge_index, line_graph_batch_mapping):
    n = x.shape[0]
    h = x @ W_emb_atom.T + b_emb_atom
    lx = line_graph_x @ W_emb_line.T + b_emb_line
    src = edge_index[0]
    dst = edge_index[1]
    ea = edge_attr[:, 0]
    l_src = line_graph_edge_index[0]
    l_dst = line_graph_edge_index[1]
    lea = line_graph_edge_attr[:, 0]
    ones_e = jnp.ones((_E,), jnp.float32)
    bc = jnp.maximum(jax.ops.segment_sum(ones_e, src, num_segments=n), 1.0)
    a_indeg = jax.ops.segment_sum(ones_e, dst, num_segments=n)
    l_indeg = jax.ops.segment_sum(jnp.ones((_LE,), jnp.float32),
                                  line_graph_edge_index[1], num_segments=_LN)

    for i in range(_NJ):
        h = _conv_atom(h, dst, src, ea, a_msg_W1[i], a_msg_b1[i], a_msg_g[i],
                       a_msg_be[i], a_msg_W2[i], a_msg_b2[i], a_upd_W[i],
                       a_upd_b[i], a_upd_g[i], a_upd_be[i], a_indeg)
        lx = _conv_line(lx, l_dst, l_src, lea,
                        l_msg_W1[i], l_msg_b1[i], l_msg_g[i], l_msg_be[i],
                        l_msg_W2[i], l_msg_b2[i], l_upd_W[i], l_upd_b[i],
                        l_upd_g[i], l_upd_be[i], l_indeg)
        abfp = _abf_scatter(lx, src, jnp.zeros((2 * _H,), jnp.float32))[0]
        abf = (abfp[0] + abfp[1])[:_N] / bc[:, None]
        hb = jnp.concatenate([h, abf], axis=1) @ b2a_W[i].T + b2a_b[i]
        h = _silu(_bn(hb, b2a_g[i], b2a_be[i]))
    for i in range(_NJ, _ANL):
        h = _conv_atom(h, dst, src, ea, a_msg_W1[i], a_msg_b1[i], a_msg_g[i],
                       a_msg_be[i], a_msg_W2[i], a_msg_b2[i], a_upd_W[i],
                       a_upd_b[i], a_upd_g[i], a_upd_be[i], a_indeg)

    atom_emb = _seg_mean(h, batch, _NB)
    line_batch = batch[line_graph_batch_mapping]
    line_emb = _seg_mean(lx, line_batch, _NB)
    g = jnp.concatenate([atom_emb, line_emb], axis=1)

    return pl.pallas_call(
        _out_kernel,
        out_shape=jax.ShapeDtypeStruct((_NB, _H), jnp.float32),
    )(g, out_W.T, out_b[None, :])
